# Initial kernel scaffold; baseline (speedup 1.0000x reference)
import jax
import jax.numpy as jnp
from jax.experimental import pallas as pl


def kernel(x, edge_index, W1, b1, W2, b2, W3, b3, Wc1, bc1, Wc2, bc2, Wc3, bc3, Wr1, br1, Wr2, br2, Wr3, br3):
    raise NotImplementedError("write your pallas kernel here")


# speedup vs baseline: 23.2429x; 23.2429x over previous
import functools

import numpy as np
import jax
import jax.numpy as jnp
from jax import lax
from jax.experimental import pallas as pl
from jax.experimental.pallas import tpu as pltpu
from jax.experimental.pallas import tpu_sc as plsc

N = 10000
D = 128
OUT = 64
E = 320000
NC = 2
NS = 16
NW = NC * NS
K = 128
CP = 80
E_PAD = NW * CP * K
N_ACC = 10112
ZPT = N_ACC // NS
OPT = N // NS
CBN = float(1.0 / np.sqrt(1.0 + 1e-5))
BM = 1000


def _worker_id():
  return lax.axis_index("s") * NC + lax.axis_index("c")


def _make_deg_kernel():
  mesh = plsc.VectorSubcoreMesh(core_axis_name="c", subcore_axis_name="s")

  def body(dst_hbm, zeros_hbm, out_hbm, dst_v, ones_v, zbuf, acc, _sem):
    cid = lax.axis_index("c")
    sid = lax.axis_index("s")
    wid = _worker_id()
    pltpu.sync_copy(dst_hbm.at[wid], dst_v)
    for i in range(K // 16):
      ones_v[pl.ds(i * 16, 16)] = jnp.full((16,), 1.0, jnp.float32)
    pltpu.sync_copy(zeros_hbm.at[pl.ds(sid * ZPT, ZPT)], zbuf)
    pltpu.sync_copy(zbuf, acc.at[pl.ds(sid * ZPT, ZPT)])
    plsc.subcore_barrier()

    def step(c, carry):
      pltpu.sync_copy(ones_v, acc.at[dst_v.at[c]], add=True)
      return carry

    lax.fori_loop(0, CP, step, 0)
    plsc.subcore_barrier()
    pltpu.sync_copy(acc.at[pl.ds(sid * ZPT, ZPT)], zbuf)
    pltpu.sync_copy(zbuf, out_hbm.at[pl.ds(cid * N_ACC + sid * ZPT, ZPT)])

  return pl.kernel(
      body,
      out_type=jax.ShapeDtypeStruct((NC * N_ACC,), jnp.float32),
      mesh=mesh,
      scratch_types=[
          pltpu.VMEM((CP, K), jnp.int32),
          pltpu.VMEM((K,), jnp.float32),
          pltpu.VMEM((ZPT,), jnp.float32),
          pltpu.VMEM_SHARED((N_ACC,), jnp.float32),
          pltpu.SemaphoreType.DMA,
      ],
  )


def _make_agg_kernel(d):
  mesh = plsc.VectorSubcoreMesh(core_axis_name="c", subcore_axis_name="s")

  def body(hw_hbm, src_hbm, dst_hbm, zeros_hbm, out_hbm,
           sbuf0, sbuf1, dbuf0, dbuf1, rows0, rows1, acc,
           isem0, isem1, gsem0, gsem1):
    cid = lax.axis_index("c")
    sid = lax.axis_index("s")
    wid = _worker_id()
    rows = (rows0, rows1)
    sbuf = (sbuf0, sbuf1)
    dbuf = (dbuf0, dbuf1)
    isem = (isem0, isem1)
    gsem = (gsem0, gsem1)

    off = sid * ZPT
    for p in range(-(-ZPT // K)):
      o = p * K
      sz = min(K, ZPT - o)
      pltpu.sync_copy(zeros_hbm.at[pl.ds(off + o, sz)],
                      rows[p % 2].at[pl.ds(0, sz)])
      pltpu.sync_copy(rows[p % 2].at[pl.ds(0, sz)],
                      acc.at[pl.ds(off + o, sz)])
    plsc.subcore_barrier()

    def fetch_idx(c, b):
      pltpu.async_copy(src_hbm.at[wid, c], sbuf[b], isem[b])
      pltpu.async_copy(dst_hbm.at[wid, c], dbuf[b], isem[b])

    def wait_idx(c, b):
      pltpu.make_async_copy(src_hbm.at[wid, c], sbuf[b], isem[b]).wait()
      pltpu.make_async_copy(dst_hbm.at[wid, c], dbuf[b], isem[b]).wait()

    def fetch_rows(b):
      pltpu.async_copy(hw_hbm.at[sbuf[b]], rows[b], gsem[b])

    fetch_idx(0, 0)
    wait_idx(0, 0)
    fetch_rows(0)
    fetch_idx(1, 1)

    def step(c0, carry):
      for b in range(2):
        c = c0 * 2 + b
        nb = 1 - b

        @pl.when(c + 1 < CP)
        def _launch_next():
          wait_idx(c + 1, nb)
          fetch_rows(nb)
        pltpu.make_async_copy(hw_hbm.at[sbuf[b]], rows[b], gsem[b]).wait()
        pltpu.sync_copy(rows[b], acc.at[dbuf[b]], add=True)

        @pl.when(c + 2 < CP)
        def _prefetch_idx():
          fetch_idx(c + 2, b)
      return carry

    lax.fori_loop(0, CP // 2, step, 0)
    plsc.subcore_barrier()
    for p in range(-(-ZPT // K)):
      o = p * K
      sz = min(K, ZPT - o)
      pltpu.sync_copy(acc.at[pl.ds(sid * ZPT + o, sz)],
                      rows[p % 2].at[pl.ds(0, sz)])
      pltpu.sync_copy(rows[p % 2].at[pl.ds(0, sz)],
                      out_hbm.at[cid, pl.ds(sid * ZPT + o, sz)])

  return pl.kernel(
      body,
      out_type=jax.ShapeDtypeStruct((NC, N_ACC, d), jnp.float32),
      mesh=mesh,
      scratch_types=[
          pltpu.VMEM((K,), jnp.int32),
          pltpu.VMEM((K,), jnp.int32),
          pltpu.VMEM((K,), jnp.int32),
          pltpu.VMEM((K,), jnp.int32),
          pltpu.VMEM((K, d), jnp.float32),
          pltpu.VMEM((K, d), jnp.float32),
          pltpu.VMEM_SHARED((N_ACC, d), jnp.float32),
          pltpu.SemaphoreType.DMA,
          pltpu.SemaphoreType.DMA,
          pltpu.SemaphoreType.DMA,
          pltpu.SemaphoreType.DMA,
      ],
  )


_deg_kernel = _make_deg_kernel()
_agg128 = _make_agg_kernel(D)


def _tc1_body(x_ref, d0_ref, d1_ref, w_ref, dinv_ref, hw_ref):
  dsum = d0_ref[...] + d1_ref[...] + 1.0
  dinv = lax.rsqrt(jnp.maximum(dsum, 1e-12))
  dinv_ref[...] = dinv
  hw_ref[...] = dinv * jnp.dot(x_ref[...], w_ref[...],
                               preferred_element_type=jnp.float32)


def _tc1(x, d0, d1, w1):
  return pl.pallas_call(
      _tc1_body,
      grid=(N // BM,),
      in_specs=[
          pl.BlockSpec((BM, D), lambda i: (i, 0)),
          pl.BlockSpec((BM, 1), lambda i: (i, 0)),
          pl.BlockSpec((BM, 1), lambda i: (i, 0)),
          pl.BlockSpec((D, D), lambda i: (0, 0)),
      ],
      out_specs=[
          pl.BlockSpec((BM, 1), lambda i: (i, 0)),
          pl.BlockSpec((BM, D), lambda i: (i, 0)),
      ],
      out_shape=[
          jax.ShapeDtypeStruct((N, 1), jnp.float32),
          jax.ShapeDtypeStruct((N, D), jnp.float32),
      ],
  )(x, d0, d1, w1)


def _tc_mid_body(agg0_ref, agg1_ref, hwp_ref, res_ref, dinv_ref, w_ref, b_ref,
                 h_ref, hwn_ref):
  dinv = dinv_ref[...]
  g = dinv * (agg0_ref[0] + agg1_ref[0] + hwp_ref[...]) + b_ref[...]
  h = jnp.maximum(res_ref[...] + CBN * g, 0.0)
  h_ref[...] = h
  hwn_ref[...] = dinv * jnp.dot(h, w_ref[...],
                                preferred_element_type=jnp.float32)


def _tc_mid(agg, hwp, res, dinv, w, b, d_out):
  return pl.pallas_call(
      _tc_mid_body,
      grid=(N // BM,),
      in_specs=[
          pl.BlockSpec((1, BM, D), lambda i: (0, i, 0)),
          pl.BlockSpec((1, BM, D), lambda i: (1, i, 0)),
          pl.BlockSpec((BM, D), lambda i: (i, 0)),
          pl.BlockSpec((BM, D), lambda i: (i, 0)),
          pl.BlockSpec((BM, 1), lambda i: (i, 0)),
          pl.BlockSpec((D, d_out), lambda i: (0, 0)),
          pl.BlockSpec((1, D), lambda i: (0, 0)),
      ],
      out_specs=[
          pl.BlockSpec((BM, D), lambda i: (i, 0)),
          pl.BlockSpec((BM, d_out), lambda i: (i, 0)),
      ],
      out_shape=[
          jax.ShapeDtypeStruct((N, D), jnp.float32),
          jax.ShapeDtypeStruct((N, d_out), jnp.float32),
      ],
  )(agg, agg, hwp, res, dinv, w, b)


def _tc_head_body(agg0_ref, agg1_ref, hwp_ref, dinv_ref, b3_ref,
                  wc1_ref, bc1_ref, wc2_ref, bc2_ref, wc3_ref, bc3_ref,
                  wr1_ref, br1_ref, wr2_ref, br2_ref, wr3_ref, br3_ref,
                  logits_ref, reg_ref, psum, pmax):
  i = pl.program_id(0)

  @pl.when(i == 0)
  def _init():
    psum[...] = jnp.zeros((1, OUT), jnp.float32)
    pmax[...] = jnp.full((1, OUT), -jnp.inf, jnp.float32)

  dinv = dinv_ref[...]
  s = (agg0_ref[0] + agg1_ref[0] + hwp_ref[...])[:, :OUT]
  h3 = CBN * (dinv * s + b3_ref[...])
  psum[...] += jnp.sum(h3, axis=0, keepdims=True)
  pmax[...] = jnp.maximum(pmax[...], jnp.max(h3, axis=0, keepdims=True))

  @pl.when(i == N // BM - 1)
  def _heads():
    g = (psum[...] / N + pmax[...]) / 2.0
    gb = jnp.broadcast_to(g, (8, OUT))

    def dot(a, w):
      return jnp.dot(a, w[...], preferred_element_type=jnp.float32)

    z = jnp.maximum(CBN * (dot(gb, wc1_ref) + bc1_ref[...]), 0.0)
    z = jnp.maximum(CBN * (dot(z, wc2_ref) + bc2_ref[...]), 0.0)
    logits_ref[...] = (dot(z, wc3_ref) + bc3_ref[...])[0:1, :]
    r = jnp.maximum(CBN * (dot(gb, wr1_ref) + br1_ref[...]), 0.0)
    r = jnp.maximum(CBN * (dot(r, wr2_ref) + br2_ref[...]), 0.0)
    reg_ref[...] = jax.nn.sigmoid((dot(r, wr3_ref) + br3_ref[...])[0:1, :])


def _tc_head(agg, hwp, dinv, b3, wc1, bc1, wc2, bc2, wc3, bc3,
             wr1, br1, wr2, br2, wr3, br3):
  full = lambda s: pl.BlockSpec(s, lambda i: tuple(0 for _ in s))
  return pl.pallas_call(
      _tc_head_body,
      grid=(N // BM,),
      in_specs=[
          pl.BlockSpec((1, BM, D), lambda i: (0, i, 0)),
          pl.BlockSpec((1, BM, D), lambda i: (1, i, 0)),
          pl.BlockSpec((BM, D), lambda i: (i, 0)),
          pl.BlockSpec((BM, 1), lambda i: (i, 0)),
          full((1, OUT)),
          full((OUT, 32)), full((1, 32)),
          full((32, 16)), full((1, 16)),
          full((16, 10)), full((1, 10)),
          full((OUT, 32)), full((1, 32)),
          full((32, 16)), full((1, 16)),
          full((16, 1)), full((1, 1)),
      ],
      out_specs=[
          pl.BlockSpec((1, 10), lambda i: (0, 0)),
          pl.BlockSpec((1, 1), lambda i: (0, 0)),
      ],
      out_shape=[
          jax.ShapeDtypeStruct((1, 10), jnp.float32),
          jax.ShapeDtypeStruct((1, 1), jnp.float32),
      ],
      scratch_shapes=[
          pltpu.VMEM((1, OUT), jnp.float32),
          pltpu.VMEM((1, OUT), jnp.float32),
      ],
  )(agg, agg, hwp, dinv, b3, wc1, bc1, wc2, bc2, wc3, bc3,
    wr1, br1, wr2, br2, wr3, br3)


def kernel(x, edge_index, W1, b1, W2, b2, W3, b3, Wc1, bc1, Wc2, bc2, Wc3,
           bc3, Wr1, br1, Wr2, br2, Wr3, br3):
  src = edge_index[0]
  dst = edge_index[1]
  pad = E_PAD - E
  ar = jnp.arange(pad, dtype=jnp.int32)
  srcp = jnp.concatenate([src, ar % np.int32(N)]).reshape(NW, CP, K)
  dstp = jnp.concatenate(
      [dst, np.int32(N) + ar % np.int32(N_ACC - N)]).reshape(NW, CP, K)

  zeros1 = jnp.zeros((N_ACC,), jnp.float32)
  zeros_d = jnp.zeros((N_ACC, D), jnp.float32)

  degp = _deg_kernel(dstp, zeros1)
  d0 = degp[:N].reshape(N, 1)
  d1 = degp[N_ACC:N_ACC + N].reshape(N, 1)

  dinv, hw1p = _tc1(x, d0, d1, W1)
  a1 = _agg128(hw1p, srcp, dstp, zeros_d)
  h1, hw2p = _tc_mid(a1, hw1p, x, dinv, W2, b1.reshape(1, D), D)
  a2 = _agg128(hw2p, srcp, dstp, zeros_d)
  w3p = jnp.pad(W3, ((0, 0), (0, D - OUT)))
  _, hw3p = _tc_mid(a2, hw2p, h1, dinv, w3p, b2.reshape(1, D), D)
  a3 = _agg128(hw3p, srcp, dstp, zeros_d)
  logits, reg = _tc_head(
      a3, hw3p, dinv, b3.reshape(1, OUT),
      Wc1, bc1.reshape(1, 32), Wc2, bc2.reshape(1, 16), Wc3, bc3.reshape(1, 10),
      Wr1, br1.reshape(1, 32), Wr2, br2.reshape(1, 16), Wr3, br3.reshape(1, 1))
  return (logits, reg)


# speedup vs baseline: 25.6471x; 1.1034x over previous
import functools

import numpy as np
import jax
import jax.numpy as jnp
from jax import lax
from jax.experimental import pallas as pl
from jax.experimental.pallas import tpu as pltpu
from jax.experimental.pallas import tpu_sc as plsc

N = 10000
D = 128
OUT = 64
E = 320000
NC = 2
NS = 16
NW = NC * NS
K = 128
CP = 80
E_PAD = NW * CP * K
N_ACC = 10112
ZPT = N_ACC // NS
OPT = N // NS
CBN = float(1.0 / np.sqrt(1.0 + 1e-5))
BM = 1000


def _worker_id():
  return lax.axis_index("s") * NC + lax.axis_index("c")


def _make_deg_kernel():
  mesh = plsc.VectorSubcoreMesh(core_axis_name="c", subcore_axis_name="s")

  def body(dst_hbm, zeros_hbm, out_hbm, dst_v, ones_v, zbuf, acc, _sem):
    cid = lax.axis_index("c")
    sid = lax.axis_index("s")
    wid = _worker_id()
    pltpu.sync_copy(dst_hbm.at[wid], dst_v)
    for i in range(K // 16):
      ones_v[pl.ds(i * 16, 16)] = jnp.full((16,), 1.0, jnp.float32)
    pltpu.sync_copy(zeros_hbm.at[pl.ds(sid * ZPT, ZPT)], zbuf)
    pltpu.sync_copy(zbuf, acc.at[pl.ds(sid * ZPT, ZPT)])
    plsc.subcore_barrier()

    def step(c, carry):
      pltpu.sync_copy(ones_v, acc.at[dst_v.at[c]], add=True)
      return carry

    lax.fori_loop(0, CP, step, 0)
    plsc.subcore_barrier()
    pltpu.sync_copy(acc.at[pl.ds(sid * ZPT, ZPT)], zbuf)
    pltpu.sync_copy(zbuf, out_hbm.at[pl.ds(cid * N_ACC + sid * ZPT, ZPT)])

  return pl.kernel(
      body,
      out_type=jax.ShapeDtypeStruct((NC * N_ACC,), jnp.float32),
      mesh=mesh,
      scratch_types=[
          pltpu.VMEM((CP, K), jnp.int32),
          pltpu.VMEM((K,), jnp.float32),
          pltpu.VMEM((ZPT,), jnp.float32),
          pltpu.VMEM_SHARED((N_ACC,), jnp.float32),
          pltpu.SemaphoreType.DMA,
      ],
  )


def _make_agg_kernel(d):
  mesh = plsc.VectorSubcoreMesh(core_axis_name="c", subcore_axis_name="s")

  def body(hw_hbm, src_hbm, dst_hbm, zeros_hbm, out_hbm,
           sbuf0, sbuf1, sbuf2, sbuf3, dbuf0, dbuf1, dbuf2, dbuf3,
           rows0, rows1, acc,
           isem0, isem1, isem2, isem3, gsem0, gsem1, ssem0, ssem1):
    cid = lax.axis_index("c")
    sid = lax.axis_index("s")
    wid = _worker_id()
    rows = (rows0, rows1)
    sbuf = (sbuf0, sbuf1, sbuf2, sbuf3)
    dbuf = (dbuf0, dbuf1, dbuf2, dbuf3)
    isem = (isem0, isem1, isem2, isem3)
    gsem = (gsem0, gsem1)
    ssem = (ssem0, ssem1)

    off = sid * ZPT
    for p in range(-(-ZPT // K)):
      o = p * K
      sz = min(K, ZPT - o)
      pltpu.sync_copy(zeros_hbm.at[pl.ds(off + o, sz)],
                      rows[p % 2].at[pl.ds(0, sz)])
      pltpu.sync_copy(rows[p % 2].at[pl.ds(0, sz)],
                      acc.at[pl.ds(off + o, sz)])
    plsc.subcore_barrier()

    def fetch_idx(c, s):
      pltpu.async_copy(src_hbm.at[wid, c], sbuf[s], isem[s])
      pltpu.async_copy(dst_hbm.at[wid, c], dbuf[s], isem[s])

    def wait_idx(c, s):
      pltpu.make_async_copy(src_hbm.at[wid, c], sbuf[s], isem[s]).wait()
      pltpu.make_async_copy(dst_hbm.at[wid, c], dbuf[s], isem[s]).wait()

    def fetch_rows(s, rb):
      pltpu.async_copy(hw_hbm.at[sbuf[s]], rows[rb], gsem[rb])

    for c in range(3):
      fetch_idx(c, c)
    wait_idx(0, 0)
    fetch_rows(0, 0)

    def step(c0, carry):
      for u in range(4):
        c = c0 * 4 + u
        rb = u % 2
        nrb = 1 - rb

        @pl.when(c + 1 < CP)
        def _launch_next():
          wait_idx(c + 1, (u + 1) % 4)

          @pl.when(c >= 1)
          def _drain_prev():
            pltpu.make_async_copy(rows[nrb], acc.at[dbuf[(u - 1) % 4]],
                                  ssem[nrb]).wait()
          fetch_rows((u + 1) % 4, nrb)

          @pl.when(c + 3 < CP)
          def _prefetch_idx():
            fetch_idx(c + 3, (u + 3) % 4)
        pltpu.make_async_copy(hw_hbm.at[sbuf[u]], rows[rb],
                              gsem[rb]).wait()
        pltpu.async_copy(rows[rb], acc.at[dbuf[u]], ssem[rb], add=True)
      return carry

    lax.fori_loop(0, CP // 4, step, 0)
    pltpu.make_async_copy(rows[0], acc.at[dbuf[0]], ssem[0]).wait()
    pltpu.make_async_copy(rows[1], acc.at[dbuf[1]], ssem[1]).wait()
    plsc.subcore_barrier()
    for p in range(-(-ZPT // K)):
      o = p * K
      sz = min(K, ZPT - o)
      pltpu.sync_copy(acc.at[pl.ds(sid * ZPT + o, sz)],
                      rows[p % 2].at[pl.ds(0, sz)])
      pltpu.sync_copy(rows[p % 2].at[pl.ds(0, sz)],
                      out_hbm.at[cid, pl.ds(sid * ZPT + o, sz)])

  return pl.kernel(
      body,
      out_type=jax.ShapeDtypeStruct((NC, N_ACC, d), jnp.float32),
      mesh=mesh,
      scratch_types=(
          [pltpu.VMEM((K,), jnp.int32)] * 8
          + [pltpu.VMEM((K, d), jnp.float32)] * 2
          + [pltpu.VMEM_SHARED((N_ACC, d), jnp.float32)]
          + [pltpu.SemaphoreType.DMA] * 8
      ),
  )


_deg_kernel = _make_deg_kernel()
_agg128 = _make_agg_kernel(D)


def _tc1_body(x_ref, d0_ref, d1_ref, w_ref, dinv_ref, hw_ref):
  dsum = d0_ref[...] + d1_ref[...] + 1.0
  dinv = lax.rsqrt(jnp.maximum(dsum, 1e-12))
  dinv_ref[...] = dinv
  hw_ref[...] = dinv * jnp.dot(x_ref[...], w_ref[...],
                               preferred_element_type=jnp.float32)


def _tc1(x, d0, d1, w1):
  return pl.pallas_call(
      _tc1_body,
      grid=(N // BM,),
      in_specs=[
          pl.BlockSpec((BM, D), lambda i: (i, 0)),
          pl.BlockSpec((BM, 1), lambda i: (i, 0)),
          pl.BlockSpec((BM, 1), lambda i: (i, 0)),
          pl.BlockSpec((D, D), lambda i: (0, 0)),
      ],
      out_specs=[
          pl.BlockSpec((BM, 1), lambda i: (i, 0)),
          pl.BlockSpec((BM, D), lambda i: (i, 0)),
      ],
      out_shape=[
          jax.ShapeDtypeStruct((N, 1), jnp.float32),
          jax.ShapeDtypeStruct((N, D), jnp.float32),
      ],
  )(x, d0, d1, w1)


def _tc_mid_body(agg0_ref, agg1_ref, hwp_ref, res_ref, dinv_ref, w_ref, b_ref,
                 h_ref, hwn_ref):
  dinv = dinv_ref[...]
  g = dinv * (agg0_ref[0] + agg1_ref[0] + hwp_ref[...]) + b_ref[...]
  h = jnp.maximum(res_ref[...] + CBN * g, 0.0)
  h_ref[...] = h
  hwn_ref[...] = dinv * jnp.dot(h, w_ref[...],
                                preferred_element_type=jnp.float32)


def _tc_mid(agg, hwp, res, dinv, w, b, d_out):
  return pl.pallas_call(
      _tc_mid_body,
      grid=(N // BM,),
      in_specs=[
          pl.BlockSpec((1, BM, D), lambda i: (0, i, 0)),
          pl.BlockSpec((1, BM, D), lambda i: (1, i, 0)),
          pl.BlockSpec((BM, D), lambda i: (i, 0)),
          pl.BlockSpec((BM, D), lambda i: (i, 0)),
          pl.BlockSpec((BM, 1), lambda i: (i, 0)),
          pl.BlockSpec((D, d_out), lambda i: (0, 0)),
          pl.BlockSpec((1, D), lambda i: (0, 0)),
      ],
      out_specs=[
          pl.BlockSpec((BM, D), lambda i: (i, 0)),
          pl.BlockSpec((BM, d_out), lambda i: (i, 0)),
      ],
      out_shape=[
          jax.ShapeDtypeStruct((N, D), jnp.float32),
          jax.ShapeDtypeStruct((N, d_out), jnp.float32),
      ],
  )(agg, agg, hwp, res, dinv, w, b)


def _tc_head_body(agg0_ref, agg1_ref, hwp_ref, dinv_ref, b3_ref,
                  wc1_ref, bc1_ref, wc2_ref, bc2_ref, wc3_ref, bc3_ref,
                  wr1_ref, br1_ref, wr2_ref, br2_ref, wr3_ref, br3_ref,
                  logits_ref, reg_ref, psum, pmax):
  i = pl.program_id(0)

  @pl.when(i == 0)
  def _init():
    psum[...] = jnp.zeros((1, OUT), jnp.float32)
    pmax[...] = jnp.full((1, OUT), -jnp.inf, jnp.float32)

  dinv = dinv_ref[...]
  s = (agg0_ref[0] + agg1_ref[0] + hwp_ref[...])[:, :OUT]
  h3 = CBN * (dinv * s + b3_ref[...])
  psum[...] += jnp.sum(h3, axis=0, keepdims=True)
  pmax[...] = jnp.maximum(pmax[...], jnp.max(h3, axis=0, keepdims=True))

  @pl.when(i == N // BM - 1)
  def _heads():
    g = (psum[...] / N + pmax[...]) / 2.0
    gb = jnp.broadcast_to(g, (8, OUT))

    def dot(a, w):
      return jnp.dot(a, w[...], preferred_element_type=jnp.float32)

    z = jnp.maximum(CBN * (dot(gb, wc1_ref) + bc1_ref[...]), 0.0)
    z = jnp.maximum(CBN * (dot(z, wc2_ref) + bc2_ref[...]), 0.0)
    logits_ref[...] = (dot(z, wc3_ref) + bc3_ref[...])[0:1, :]
    r = jnp.maximum(CBN * (dot(gb, wr1_ref) + br1_ref[...]), 0.0)
    r = jnp.maximum(CBN * (dot(r, wr2_ref) + br2_ref[...]), 0.0)
    reg_ref[...] = jax.nn.sigmoid((dot(r, wr3_ref) + br3_ref[...])[0:1, :])


def _tc_head(agg, hwp, dinv, b3, wc1, bc1, wc2, bc2, wc3, bc3,
             wr1, br1, wr2, br2, wr3, br3):
  full = lambda s: pl.BlockSpec(s, lambda i: tuple(0 for _ in s))
  return pl.pallas_call(
      _tc_head_body,
      grid=(N // BM,),
      in_specs=[
          pl.BlockSpec((1, BM, D), lambda i: (0, i, 0)),
          pl.BlockSpec((1, BM, D), lambda i: (1, i, 0)),
          pl.BlockSpec((BM, D), lambda i: (i, 0)),
          pl.BlockSpec((BM, 1), lambda i: (i, 0)),
          full((1, OUT)),
          full((OUT, 32)), full((1, 32)),
          full((32, 16)), full((1, 16)),
          full((16, 10)), full((1, 10)),
          full((OUT, 32)), full((1, 32)),
          full((32, 16)), full((1, 16)),
          full((16, 1)), full((1, 1)),
      ],
      out_specs=[
          pl.BlockSpec((1, 10), lambda i: (0, 0)),
          pl.BlockSpec((1, 1), lambda i: (0, 0)),
      ],
      out_shape=[
          jax.ShapeDtypeStruct((1, 10), jnp.float32),
          jax.ShapeDtypeStruct((1, 1), jnp.float32),
      ],
      scratch_shapes=[
          pltpu.VMEM((1, OUT), jnp.float32),
          pltpu.VMEM((1, OUT), jnp.float32),
      ],
  )(agg, agg, hwp, dinv, b3, wc1, bc1, wc2, bc2, wc3, bc3,
    wr1, br1, wr2, br2, wr3, br3)


def kernel(x, edge_index, W1, b1, W2, b2, W3, b3, Wc1, bc1, Wc2, bc2, Wc3,
           bc3, Wr1, br1, Wr2, br2, Wr3, br3):
  src = edge_index[0]
  dst = edge_index[1]
  pad = E_PAD - E
  ar = jnp.arange(pad, dtype=jnp.int32)
  srcp = jnp.concatenate([src, ar % np.int32(N)]).reshape(NW, CP, K)
  dstp = jnp.concatenate(
      [dst, np.int32(N) + ar % np.int32(N_ACC - N)]).reshape(NW, CP, K)

  zeros1 = jnp.zeros((N_ACC,), jnp.float32)
  zeros_d = jnp.zeros((N_ACC, D), jnp.float32)

  degp = _deg_kernel(dstp, zeros1)
  d0 = degp[:N].reshape(N, 1)
  d1 = degp[N_ACC:N_ACC + N].reshape(N, 1)

  dinv, hw1p = _tc1(x, d0, d1, W1)
  a1 = _agg128(hw1p, srcp, dstp, zeros_d)
  h1, hw2p = _tc_mid(a1, hw1p, x, dinv, W2, b1.reshape(1, D), D)
  a2 = _agg128(hw2p, srcp, dstp, zeros_d)
  w3p = jnp.pad(W3, ((0, 0), (0, D - OUT)))
  _, hw3p = _tc_mid(a2, hw2p, h1, dinv, w3p, b2.reshape(1, D), D)
  a3 = _agg128(hw3p, srcp, dstp, zeros_d)
  logits, reg = _tc_head(
      a3, hw3p, dinv, b3.reshape(1, OUT),
      Wc1, bc1.reshape(1, 32), Wc2, bc2.reshape(1, 16), Wc3, bc3.reshape(1, 10),
      Wr1, br1.reshape(1, 32), Wr2, br2.reshape(1, 16), Wr3, br3.reshape(1, 1))
  return (logits, reg)


# speedup vs baseline: 27.3105x; 1.0649x over previous
import functools

import numpy as np
import jax
import jax.numpy as jnp
from jax import lax
from jax.experimental import pallas as pl
from jax.experimental.pallas import tpu as pltpu
from jax.experimental.pallas import tpu_sc as plsc

N = 10000
D = 128
OUT = 64
E = 320000
NC = 2
NS = 16
NW = NC * NS
K = 128
CP = 80
E_PAD = NW * CP * K
N_ACC = 10112
ZPT = N_ACC // NS
OPT = N // NS
CBN = float(1.0 / np.sqrt(1.0 + 1e-5))
BM = 2000


def _worker_id():
  return lax.axis_index("s") * NC + lax.axis_index("c")


def _make_deg_kernel():
  mesh = plsc.VectorSubcoreMesh(core_axis_name="c", subcore_axis_name="s")

  def body(dst_hbm, zeros_hbm, out_hbm, dst_v, ones_v, zbuf, acc, _sem):
    cid = lax.axis_index("c")
    sid = lax.axis_index("s")
    wid = _worker_id()
    pltpu.sync_copy(dst_hbm.at[wid], dst_v)
    for i in range(K // 16):
      ones_v[pl.ds(i * 16, 16)] = jnp.full((16,), 1.0, jnp.float32)
    pltpu.sync_copy(zeros_hbm, zbuf)
    pltpu.sync_copy(zbuf, acc.at[pl.ds(sid * ZPT, ZPT)])
    plsc.subcore_barrier()

    def step(c, carry):
      pltpu.sync_copy(ones_v, acc.at[dst_v.at[c]], add=True)
      return carry

    lax.fori_loop(0, CP, step, 0)
    plsc.subcore_barrier()
    pltpu.sync_copy(acc.at[pl.ds(sid * ZPT, ZPT)], zbuf)
    pltpu.sync_copy(zbuf, out_hbm.at[pl.ds(cid * N_ACC + sid * ZPT, ZPT)])

  return pl.kernel(
      body,
      out_type=jax.ShapeDtypeStruct((NC * N_ACC,), jnp.float32),
      mesh=mesh,
      scratch_types=[
          pltpu.VMEM((CP, K), jnp.int32),
          pltpu.VMEM((K,), jnp.float32),
          pltpu.VMEM((ZPT,), jnp.float32),
          pltpu.VMEM_SHARED((N_ACC,), jnp.float32),
          pltpu.SemaphoreType.DMA,
      ],
  )


def _make_agg_kernel(d):
  mesh = plsc.VectorSubcoreMesh(core_axis_name="c", subcore_axis_name="s")

  def body(hw_hbm, src_hbm, dst_hbm, zeros_hbm, out_hbm,
           sbuf0, sbuf1, sbuf2, sbuf3, dbuf0, dbuf1, dbuf2, dbuf3,
           rows0, rows1, acc,
           isem0, isem1, isem2, isem3, gsem0, gsem1, ssem0, ssem1):
    cid = lax.axis_index("c")
    sid = lax.axis_index("s")
    wid = _worker_id()
    rows = (rows0, rows1)
    sbuf = (sbuf0, sbuf1, sbuf2, sbuf3)
    dbuf = (dbuf0, dbuf1, dbuf2, dbuf3)
    isem = (isem0, isem1, isem2, isem3)
    gsem = (gsem0, gsem1)
    ssem = (ssem0, ssem1)

    off = sid * ZPT
    pltpu.sync_copy(zeros_hbm, rows[0])
    for p in range(-(-ZPT // K)):
      o = p * K
      sz = min(K, ZPT - o)
      pltpu.sync_copy(rows[0].at[pl.ds(0, sz)], acc.at[pl.ds(off + o, sz)])
    plsc.subcore_barrier()

    def fetch_idx(c, s):
      pltpu.async_copy(src_hbm.at[wid, c], sbuf[s], isem[s])
      pltpu.async_copy(dst_hbm.at[wid, c], dbuf[s], isem[s])

    def wait_idx(c, s):
      pltpu.make_async_copy(src_hbm.at[wid, c], sbuf[s], isem[s]).wait()
      pltpu.make_async_copy(dst_hbm.at[wid, c], dbuf[s], isem[s]).wait()

    def fetch_rows(s, rb):
      pltpu.async_copy(hw_hbm.at[sbuf[s]], rows[rb], gsem[rb])

    for c in range(3):
      fetch_idx(c, c)
    wait_idx(0, 0)
    fetch_rows(0, 0)

    def step(c0, carry):
      for u in range(4):
        c = c0 * 4 + u
        rb = u % 2
        nrb = 1 - rb

        @pl.when(c + 1 < CP)
        def _launch_next():
          wait_idx(c + 1, (u + 1) % 4)

          @pl.when(c >= 1)
          def _drain_prev():
            pltpu.make_async_copy(rows[nrb], acc.at[dbuf[(u - 1) % 4]],
                                  ssem[nrb]).wait()
          fetch_rows((u + 1) % 4, nrb)

          @pl.when(c + 3 < CP)
          def _prefetch_idx():
            fetch_idx(c + 3, (u + 3) % 4)
        pltpu.make_async_copy(hw_hbm.at[sbuf[u]], rows[rb],
                              gsem[rb]).wait()
        pltpu.async_copy(rows[rb], acc.at[dbuf[u]], ssem[rb], add=True)
      return carry

    lax.fori_loop(0, CP // 4, step, 0)
    pltpu.make_async_copy(rows[0], acc.at[dbuf[0]], ssem[0]).wait()
    pltpu.make_async_copy(rows[1], acc.at[dbuf[1]], ssem[1]).wait()
    plsc.subcore_barrier()
    for p in range(-(-ZPT // K)):
      o = p * K
      sz = min(K, ZPT - o)
      pltpu.sync_copy(acc.at[pl.ds(sid * ZPT + o, sz)],
                      rows[p % 2].at[pl.ds(0, sz)])
      pltpu.sync_copy(rows[p % 2].at[pl.ds(0, sz)],
                      out_hbm.at[cid, pl.ds(sid * ZPT + o, sz)])

  return pl.kernel(
      body,
      out_type=jax.ShapeDtypeStruct((NC, N_ACC, d), jnp.float32),
      mesh=mesh,
      scratch_types=(
          [pltpu.VMEM((K,), jnp.int32)] * 8
          + [pltpu.VMEM((K, d), jnp.float32)] * 2
          + [pltpu.VMEM_SHARED((N_ACC, d), jnp.float32)]
          + [pltpu.SemaphoreType.DMA] * 8
      ),
  )


_deg_kernel = _make_deg_kernel()
_agg128 = _make_agg_kernel(D)


def _tc1_body(x_ref, ds_ref, w_ref, dinv_ref, hw_ref):
  dsum = ds_ref[...] + 1.0
  dinv = lax.rsqrt(jnp.maximum(dsum, 1e-12))
  dinv_ref[...] = dinv
  hw_ref[...] = dinv * jnp.dot(x_ref[...], w_ref[...],
                               preferred_element_type=jnp.float32)


def _tc1(x, dsum, w1):
  return pl.pallas_call(
      _tc1_body,
      grid=(N // BM,),
      in_specs=[
          pl.BlockSpec((BM, D), lambda i: (i, 0)),
          pl.BlockSpec((BM, 1), lambda i: (i, 0)),
          pl.BlockSpec((D, D), lambda i: (0, 0)),
      ],
      out_specs=[
          pl.BlockSpec((BM, 1), lambda i: (i, 0)),
          pl.BlockSpec((BM, D), lambda i: (i, 0)),
      ],
      out_shape=[
          jax.ShapeDtypeStruct((N, 1), jnp.float32),
          jax.ShapeDtypeStruct((N, D), jnp.float32),
      ],
  )(x, dsum, w1)


def _tc_mid_body(agg0_ref, agg1_ref, hwp_ref, res_ref, dinv_ref, w_ref, b_ref,
                 h_ref, hwn_ref):
  dinv = dinv_ref[...]
  g = dinv * (agg0_ref[0] + agg1_ref[0] + hwp_ref[...]) + b_ref[...]
  h = jnp.maximum(res_ref[...] + CBN * g, 0.0)
  h_ref[...] = h
  hwn_ref[...] = dinv * jnp.dot(h, w_ref[...],
                                preferred_element_type=jnp.float32)


def _tc_mid(agg, hwp, res, dinv, w, b, d_out):
  return pl.pallas_call(
      _tc_mid_body,
      grid=(N // BM,),
      in_specs=[
          pl.BlockSpec((1, BM, D), lambda i: (0, i, 0)),
          pl.BlockSpec((1, BM, D), lambda i: (1, i, 0)),
          pl.BlockSpec((BM, D), lambda i: (i, 0)),
          pl.BlockSpec((BM, D), lambda i: (i, 0)),
          pl.BlockSpec((BM, 1), lambda i: (i, 0)),
          pl.BlockSpec((D, d_out), lambda i: (0, 0)),
          pl.BlockSpec((1, D), lambda i: (0, 0)),
      ],
      out_specs=[
          pl.BlockSpec((BM, D), lambda i: (i, 0)),
          pl.BlockSpec((BM, d_out), lambda i: (i, 0)),
      ],
      out_shape=[
          jax.ShapeDtypeStruct((N, D), jnp.float32),
          jax.ShapeDtypeStruct((N, d_out), jnp.float32),
      ],
  )(agg, agg, hwp, res, dinv, w, b)


def _tc_head_body(agg0_ref, agg1_ref, hwp_ref, dinv_ref, b3_ref,
                  wc1_ref, bc1_ref, wc2_ref, bc2_ref, wc3_ref, bc3_ref,
                  wr1_ref, br1_ref, wr2_ref, br2_ref, wr3_ref, br3_ref,
                  logits_ref, reg_ref, psum, pmax):
  i = pl.program_id(0)

  @pl.when(i == 0)
  def _init():
    psum[...] = jnp.zeros((1, OUT), jnp.float32)
    pmax[...] = jnp.full((1, OUT), -jnp.inf, jnp.float32)

  dinv = dinv_ref[...]
  s = (agg0_ref[0] + agg1_ref[0] + hwp_ref[...])[:, :OUT]
  h3 = CBN * (dinv * s + b3_ref[...])
  psum[...] += jnp.sum(h3, axis=0, keepdims=True)
  pmax[...] = jnp.maximum(pmax[...], jnp.max(h3, axis=0, keepdims=True))

  @pl.when(i == N // BM - 1)
  def _heads():
    g = (psum[...] / N + pmax[...]) / 2.0
    gb = jnp.broadcast_to(g, (8, OUT))

    def dot(a, w):
      return jnp.dot(a, w[...], preferred_element_type=jnp.float32)

    z = jnp.maximum(CBN * (dot(gb, wc1_ref) + bc1_ref[...]), 0.0)
    z = jnp.maximum(CBN * (dot(z, wc2_ref) + bc2_ref[...]), 0.0)
    logits_ref[...] = (dot(z, wc3_ref) + bc3_ref[...])[0:1, :]
    r = jnp.maximum(CBN * (dot(gb, wr1_ref) + br1_ref[...]), 0.0)
    r = jnp.maximum(CBN * (dot(r, wr2_ref) + br2_ref[...]), 0.0)
    reg_ref[...] = jax.nn.sigmoid((dot(r, wr3_ref) + br3_ref[...])[0:1, :])


def _tc_head(agg, hwp, dinv, b3, wc1, bc1, wc2, bc2, wc3, bc3,
             wr1, br1, wr2, br2, wr3, br3):
  full = lambda s: pl.BlockSpec(s, lambda i: tuple(0 for _ in s))
  return pl.pallas_call(
      _tc_head_body,
      grid=(N // BM,),
      in_specs=[
          pl.BlockSpec((1, BM, D), lambda i: (0, i, 0)),
          pl.BlockSpec((1, BM, D), lambda i: (1, i, 0)),
          pl.BlockSpec((BM, D), lambda i: (i, 0)),
          pl.BlockSpec((BM, 1), lambda i: (i, 0)),
          full((1, OUT)),
          full((OUT, 32)), full((1, 32)),
          full((32, 16)), full((1, 16)),
          full((16, 10)), full((1, 10)),
          full((OUT, 32)), full((1, 32)),
          full((32, 16)), full((1, 16)),
          full((16, 1)), full((1, 1)),
      ],
      out_specs=[
          pl.BlockSpec((1, 10), lambda i: (0, 0)),
          pl.BlockSpec((1, 1), lambda i: (0, 0)),
      ],
      out_shape=[
          jax.ShapeDtypeStruct((1, 10), jnp.float32),
          jax.ShapeDtypeStruct((1, 1), jnp.float32),
      ],
      scratch_shapes=[
          pltpu.VMEM((1, OUT), jnp.float32),
          pltpu.VMEM((1, OUT), jnp.float32),
      ],
  )(agg, agg, hwp, dinv, b3, wc1, bc1, wc2, bc2, wc3, bc3,
    wr1, br1, wr2, br2, wr3, br3)


def kernel(x, edge_index, W1, b1, W2, b2, W3, b3, Wc1, bc1, Wc2, bc2, Wc3,
           bc3, Wr1, br1, Wr2, br2, Wr3, br3):
  src = edge_index[0]
  dst = edge_index[1]
  pad = E_PAD - E
  ar = jnp.arange(pad, dtype=jnp.int32)
  srcp = jnp.concatenate([src, ar % np.int32(N)]).reshape(NW, CP, K)
  dstp = jnp.concatenate(
      [dst, np.int32(N) + ar % np.int32(N_ACC - N)]).reshape(NW, CP, K)

  zeros1 = jnp.zeros((ZPT,), jnp.float32)
  zeros_d = jnp.zeros((K, D), jnp.float32)

  degp = _deg_kernel(dstp, zeros1)
  dsum = (degp[:N] + degp[N_ACC:N_ACC + N]).reshape(N, 1)
  dinv, hw1p = _tc1(x, dsum, W1)
  a1 = _agg128(hw1p, srcp, dstp, zeros_d)
  h1, hw2p = _tc_mid(a1, hw1p, x, dinv, W2, b1.reshape(1, D), D)
  a2 = _agg128(hw2p, srcp, dstp, zeros_d)
  w3p = jnp.pad(W3, ((0, 0), (0, D - OUT)))
  _, hw3p = _tc_mid(a2, hw2p, h1, dinv, w3p, b2.reshape(1, D), D)
  a3 = _agg128(hw3p, srcp, dstp, zeros_d)
  logits, reg = _tc_head(
      a3, hw3p, dinv, b3.reshape(1, OUT),
      Wc1, bc1.reshape(1, 32), Wc2, bc2.reshape(1, 16), Wc3, bc3.reshape(1, 10),
      Wr1, br1.reshape(1, 32), Wr2, br2.reshape(1, 16), Wr3, br3.reshape(1, 1))
  return (logits, reg)
